# baseline (speedup 1.0000x reference)
import functools

import jax
import jax.numpy as jnp
from jax import lax
from jax.experimental import pallas as pl
from jax.experimental.pallas import tpu as pltpu
from jax.experimental.pallas import tpu_sc as plsc

_VOCAB = 1000000
_D = 64
_MAX_WAVELENGTH = 10000.0
_B = 1024
_L = 200
_TOTAL = _B * _L

_NC = 2
_NS = 16
_NW = _NC * _NS
_PER_W = _TOTAL // _NW
_SUB = 100
_IDX_ROWS_PER_W = _PER_W // _SUB
_NBUF = 8


_PACK_T = 1024
_PACK_GRID = (_VOCAB + _PACK_T - 1) // _PACK_T


def _pack_body(x_ref, out_ref):
    out_ref[:, :_D] = x_ref[...].T


def _make_packed(table_t):
    return pl.pallas_call(
        _pack_body,
        grid=(_PACK_GRID,),
        in_specs=[pl.BlockSpec((_D, _PACK_T), lambda j: (0, j))],
        out_specs=pl.BlockSpec((_PACK_T, 2 * _D), lambda j: (j, 0)),
        out_shape=jax.ShapeDtypeStruct((_VOCAB, 2 * _D), jnp.float32),
    )(table_t)


def _enc_body(out_ref):
    pos = lax.broadcasted_iota(jnp.int32, (_L, _D), 0).astype(jnp.float32)
    i = lax.broadcasted_iota(jnp.int32, (_L, _D), 1)
    expo = (2 * (i // 2)).astype(jnp.float32) * (1.0 / _D)
    timescales = jnp.exp(expo * jnp.log(jnp.float32(1.0 / _MAX_WAVELENGTH)))
    angles = pos * timescales
    odd = (i % 2).astype(jnp.float32)
    out_ref[...] = jnp.sin(angles) * (1.0 - odd) + jnp.cos(angles) * odd


def _make_enc():
    return pl.pallas_call(
        _enc_body,
        out_shape=jax.ShapeDtypeStruct((_L, _D), jnp.float32),
    )()


def _sc_body(idx_hbm, table_hbm, enc_hbm, out_hbm,
             idx_v, enc_v, rows_v, sem):
    wid = lax.axis_index("s") * _NC + lax.axis_index("c")
    idx_row0 = wid * _IDX_ROWS_PER_W
    out_base = wid * _PER_W

    pltpu.sync_copy(idx_hbm.at[pl.ds(idx_row0, _IDX_ROWS_PER_W)], idx_v)
    pltpu.sync_copy(enc_hbm, enc_v)

    for g in range(_NBUF):
        pltpu.async_copy(table_hbm.at[idx_v.at[g]],
                         rows_v.at[pl.ds(g * _SUB, _SUB)], sem)

    @pl.loop(0, _IDX_ROWS_PER_W)
    def _step(g):
        b = lax.rem(g, _NBUF)
        row0 = b * _SUB
        pltpu.make_async_copy(table_hbm.at[idx_v.at[g]],
                              rows_v.at[pl.ds(row0, _SUB)], sem).wait()

        ph = lax.rem(g, _L // _SUB) * _SUB

        @pl.loop(0, _SUB, unroll=4)
        def _add(r):
            for d in range(_D // 16):
                plsc.addupdate(rows_v.at[row0 + r, pl.ds(d * 16, 16)],
                               enc_v[ph + r, pl.ds(d * 16, 16)])

        pltpu.sync_copy(rows_v.at[pl.ds(row0, _SUB), pl.ds(0, _D)],
                        out_hbm.at[pl.ds(out_base + g * _SUB, _SUB)])

        @pl.when(g < _IDX_ROWS_PER_W - _NBUF)
        def _refill():
            pltpu.async_copy(table_hbm.at[idx_v.at[g + _NBUF]],
                             rows_v.at[pl.ds(row0, _SUB)], sem)


_sc_gather = functools.partial(
    pl.kernel,
    out_type=jax.ShapeDtypeStruct((_TOTAL, _D), jnp.float32),
    mesh=plsc.VectorSubcoreMesh(core_axis_name="c", subcore_axis_name="s"),
    scratch_types=[
        pltpu.VMEM((_IDX_ROWS_PER_W, _SUB), jnp.int32),
        pltpu.VMEM((_L, _D), jnp.float32),
        pltpu.VMEM((_NBUF * _SUB, 2 * _D), jnp.float32),
        pltpu.SemaphoreType.DMA,
    ],
    compiler_params=pltpu.CompilerParams(use_tc_tiling_on_sc=False),
)(_sc_body)


def kernel(inputs, table):
    idx2d = inputs.reshape(_TOTAL // _SUB, _SUB).astype(jnp.int32)
    enc = _make_enc()
    packed = _make_packed(table.T)
    out = _sc_gather(idx2d, packed, enc)
    return out.reshape(_B, _L, _D)


# speedup vs baseline: 1.2296x; 1.2296x over previous
import functools

import jax
import jax.numpy as jnp
from jax import lax
from jax.experimental import pallas as pl
from jax.experimental.pallas import tpu as pltpu
from jax.experimental.pallas import tpu_sc as plsc

_VOCAB = 1000000
_D = 64
_MAX_WAVELENGTH = 10000.0
_B = 1024
_L = 200
_TOTAL = _B * _L

_NC = 2
_NS = 16
_NW = _NC * _NS
_PER_W = _TOTAL // _NW
_SUB = 100
_IDX_ROWS_PER_W = _PER_W // _SUB
_NBUF = 8


def _enc_body(out_ref):
    pos = lax.broadcasted_iota(jnp.int32, (_L, _D), 0).astype(jnp.float32)
    i = lax.broadcasted_iota(jnp.int32, (_L, _D), 1)
    expo = (2 * (i // 2)).astype(jnp.float32) * (1.0 / _D)
    timescales = jnp.exp(expo * jnp.log(jnp.float32(1.0 / _MAX_WAVELENGTH)))
    angles = pos * timescales
    odd = (i % 2).astype(jnp.float32)
    out_ref[...] = jnp.sin(angles) * (1.0 - odd) + jnp.cos(angles) * odd


def _make_enc():
    return pl.pallas_call(
        _enc_body,
        out_shape=jax.ShapeDtypeStruct((_L, _D), jnp.float32),
    )()


def _sc_body(idx_hbm, table_hbm, enc_hbm, out_hbm,
             idx_v, enc_v, rows_v, sem):
    wid = lax.axis_index("s") * _NC + lax.axis_index("c")
    idx_row0 = wid * _IDX_ROWS_PER_W
    out_base = wid * (_PER_W // _L)

    pltpu.sync_copy(idx_hbm.at[pl.ds(idx_row0, _IDX_ROWS_PER_W)], idx_v)
    pltpu.sync_copy(enc_hbm, enc_v)

    for g in range(_NBUF):
        pltpu.async_copy(table_hbm.at[idx_v.at[g]],
                         rows_v.at[pl.ds(g * _SUB, _SUB)], sem)

    @pl.loop(0, _IDX_ROWS_PER_W)
    def _step(g):
        b = lax.rem(g, _NBUF)
        row0 = b * _SUB
        pltpu.make_async_copy(table_hbm.at[idx_v.at[g]],
                              rows_v.at[pl.ds(row0, _SUB)], sem).wait()

        ph = lax.rem(g, _L // _SUB) * _SUB

        @pl.loop(0, _SUB, unroll=4)
        def _add(r):
            for d in range(_D // 16):
                plsc.addupdate(rows_v.at[row0 + r, pl.ds(d * 16, 16)],
                               enc_v[ph + r, pl.ds(d * 16, 16)])

        b = out_base + lax.div(g, _L // _SUB)
        pltpu.sync_copy(rows_v.at[pl.ds(row0, _SUB), pl.ds(0, _D)],
                        out_hbm.at[b, pl.ds(ph, _SUB)])

        @pl.when(g < _IDX_ROWS_PER_W - _NBUF)
        def _refill():
            pltpu.async_copy(table_hbm.at[idx_v.at[g + _NBUF]],
                             rows_v.at[pl.ds(row0, _SUB)], sem)


_sc_gather = functools.partial(
    pl.kernel,
    out_type=jax.ShapeDtypeStruct((_B, _L, _D), jnp.float32),
    mesh=plsc.VectorSubcoreMesh(core_axis_name="c", subcore_axis_name="s"),
    scratch_types=[
        pltpu.VMEM((_IDX_ROWS_PER_W, _SUB), jnp.int32),
        pltpu.VMEM((_L, _D), jnp.float32),
        pltpu.VMEM((_NBUF * _SUB, 2 * _D), jnp.float32),
        pltpu.SemaphoreType.DMA,
    ],
    compiler_params=pltpu.CompilerParams(use_tc_tiling_on_sc=False),
)(_sc_body)


def kernel(inputs, table):
    idx2d = inputs.reshape(_TOTAL // _SUB, _SUB).astype(jnp.int32)
    enc = _make_enc()
    table_p = jnp.pad(table, ((0, 0), (0, _D)))
    return _sc_gather(idx2d, table_p, enc)


# speedup vs baseline: 1.2318x; 1.0018x over previous
import functools

import jax
import jax.numpy as jnp
from jax import lax
from jax.experimental import pallas as pl
from jax.experimental.pallas import tpu as pltpu
from jax.experimental.pallas import tpu_sc as plsc

_VOCAB = 1000000
_D = 64
_MAX_WAVELENGTH = 10000.0
_B = 1024
_L = 200
_TOTAL = _B * _L

_NC = 2
_NS = 16
_NW = _NC * _NS
_PER_W = _TOTAL // _NW
_SUB = 100
_IDX_ROWS_PER_W = _PER_W // _SUB
_NBUF = 8


def _enc_body(out_ref):
    pos = lax.broadcasted_iota(jnp.int32, (_L, _D), 0).astype(jnp.float32)
    i = lax.broadcasted_iota(jnp.int32, (_L, _D), 1)
    expo = (2 * (i // 2)).astype(jnp.float32) * (1.0 / _D)
    timescales = jnp.exp(expo * jnp.log(jnp.float32(1.0 / _MAX_WAVELENGTH)))
    angles = pos * timescales
    odd = (i % 2).astype(jnp.float32)
    out_ref[...] = jnp.sin(angles) * (1.0 - odd) + jnp.cos(angles) * odd


def _make_enc():
    return pl.pallas_call(
        _enc_body,
        out_shape=jax.ShapeDtypeStruct((_L, _D), jnp.float32),
    )()


def _sc_body(idx_hbm, table_hbm, enc_hbm, out_hbm,
             idx_v, enc_v, rows_v, sem):
    wid = lax.axis_index("s") * _NC + lax.axis_index("c")
    idx_row0 = wid * _IDX_ROWS_PER_W
    out_base = wid * (_PER_W // _L)

    pltpu.sync_copy(idx_hbm.at[pl.ds(idx_row0, _IDX_ROWS_PER_W)], idx_v)
    pltpu.sync_copy(enc_hbm, enc_v)

    for g in range(_NBUF):
        pltpu.async_copy(table_hbm.at[idx_v.at[g]],
                         rows_v.at[pl.ds(g * _SUB, _SUB)], sem)

    @pl.loop(0, _IDX_ROWS_PER_W)
    def _step(g):
        b = lax.rem(g, _NBUF)
        row0 = b * _SUB
        pltpu.make_async_copy(table_hbm.at[idx_v.at[g]],
                              rows_v.at[pl.ds(row0, _SUB)], sem).wait()

        ph = lax.rem(g, _L // _SUB) * _SUB

        @pl.loop(0, _SUB, unroll=4)
        def _add(r):
            for d in range(_D // 16):
                plsc.addupdate(rows_v.at[row0 + r, pl.ds(d * 16, 16)],
                               enc_v[ph + r, pl.ds(d * 16, 16)])

        brow = out_base + lax.div(g, _L // _SUB)
        pltpu.sync_copy(rows_v.at[pl.ds(row0, _SUB), pl.ds(0, _D)],
                        out_hbm.at[brow, pl.ds(ph, _SUB)])

        @pl.when(g < _IDX_ROWS_PER_W - _NBUF)
        def _refill():
            pltpu.async_copy(table_hbm.at[idx_v.at[g + _NBUF]],
                             rows_v.at[pl.ds(row0, _SUB)], sem)


_sc_gather = functools.partial(
    pl.kernel,
    out_type=jax.ShapeDtypeStruct((_B, _L, _D), jnp.float32),
    mesh=plsc.VectorSubcoreMesh(core_axis_name="c", subcore_axis_name="s"),
    scratch_types=[
        pltpu.VMEM((_IDX_ROWS_PER_W, _SUB), jnp.int32),
        pltpu.VMEM((_L, _D), jnp.float32),
        pltpu.VMEM((_NBUF * _SUB, 2 * _D), jnp.float32),
        pltpu.SemaphoreType.DMA,
    ],
    compiler_params=pltpu.CompilerParams(use_tc_tiling_on_sc=False),
)(_sc_body)


def kernel(inputs, table):
    idx2d = inputs.reshape(_TOTAL // _SUB, _SUB).astype(jnp.int32)
    enc = _make_enc()
    table_p = jnp.pad(table, ((0, 0), (0, _D)))
    return _sc_gather(idx2d, table_p, enc)


# speedup vs baseline: 1.2369x; 1.0042x over previous
import functools

import jax
import jax.numpy as jnp
from jax import lax
from jax.experimental import pallas as pl
from jax.experimental.pallas import tpu as pltpu
from jax.experimental.pallas import tpu_sc as plsc

_VOCAB = 1000000
_D = 64
_MAX_WAVELENGTH = 10000.0
_B = 1024
_L = 200
_TOTAL = _B * _L

_NC = 2
_NS = 16
_NW = _NC * _NS
_PER_W = _TOTAL // _NW
_SUB = 100
_IDX_ROWS_PER_W = _PER_W // _SUB
_NBUF = 8
_DEPTH = 7


def _enc_body(out_ref):
    pos = lax.broadcasted_iota(jnp.int32, (_L, _D), 0).astype(jnp.float32)
    i = lax.broadcasted_iota(jnp.int32, (_L, _D), 1)
    expo = (2 * (i // 2)).astype(jnp.float32) * (1.0 / _D)
    timescales = jnp.exp(expo * jnp.log(jnp.float32(1.0 / _MAX_WAVELENGTH)))
    angles = pos * timescales
    odd = (i % 2).astype(jnp.float32)
    out_ref[...] = jnp.sin(angles) * (1.0 - odd) + jnp.cos(angles) * odd


def _make_enc():
    return pl.pallas_call(
        _enc_body,
        out_shape=jax.ShapeDtypeStruct((_L, _D), jnp.float32),
    )()


def _sc_body(idx_hbm, table_hbm, enc_hbm, out_hbm,
             idx_v, enc_v, rows_v, sem, wb_sem):
    wid = lax.axis_index("s") * _NC + lax.axis_index("c")
    idx_row0 = wid * _IDX_ROWS_PER_W
    out_base = wid * (_PER_W // _L)

    pltpu.sync_copy(idx_hbm.at[pl.ds(idx_row0, _IDX_ROWS_PER_W)], idx_v)
    pltpu.sync_copy(enc_hbm, enc_v)

    for g in range(_DEPTH):
        pltpu.async_copy(table_hbm.at[idx_v.at[g]],
                         rows_v.at[pl.ds(g * _SUB, _SUB)], sem)

    @pl.loop(0, _IDX_ROWS_PER_W)
    def _step(g):
        row0 = lax.rem(g, _NBUF) * _SUB
        pltpu.make_async_copy(table_hbm.at[idx_v.at[g]],
                              rows_v.at[pl.ds(row0, _SUB)], sem).wait()

        ph = lax.rem(g, _L // _SUB) * _SUB

        @pl.loop(0, _SUB, unroll=4)
        def _add(r):
            for d in range(_D // 16):
                plsc.addupdate(rows_v.at[row0 + r, pl.ds(d * 16, 16)],
                               enc_v[ph + r, pl.ds(d * 16, 16)])

        brow = out_base + lax.div(g, _L // _SUB)
        pltpu.async_copy(rows_v.at[pl.ds(row0, _SUB), pl.ds(0, _D)],
                         out_hbm.at[brow, pl.ds(ph, _SUB)], wb_sem)

        @pl.when(g > 0)
        def _drain_wb():
            pltpu.make_async_copy(rows_v.at[pl.ds(0, _SUB), pl.ds(0, _D)],
                                  out_hbm.at[0, pl.ds(0, _SUB)],
                                  wb_sem).wait()

        @pl.when(g < _IDX_ROWS_PER_W - _DEPTH)
        def _refill():
            pltpu.async_copy(table_hbm.at[idx_v.at[g + _DEPTH]],
                             rows_v.at[pl.ds(lax.rem(g + _DEPTH, _NBUF) * _SUB,
                                             _SUB)], sem)

    pltpu.make_async_copy(rows_v.at[pl.ds(0, _SUB), pl.ds(0, _D)],
                          out_hbm.at[0, pl.ds(0, _SUB)], wb_sem).wait()


_sc_gather = functools.partial(
    pl.kernel,
    out_type=jax.ShapeDtypeStruct((_B, _L, _D), jnp.float32),
    mesh=plsc.VectorSubcoreMesh(core_axis_name="c", subcore_axis_name="s"),
    scratch_types=[
        pltpu.VMEM((_IDX_ROWS_PER_W, _SUB), jnp.int32),
        pltpu.VMEM((_L, _D), jnp.float32),
        pltpu.VMEM((_NBUF * _SUB, 2 * _D), jnp.float32),
        pltpu.SemaphoreType.DMA,
        pltpu.SemaphoreType.DMA,
    ],
    compiler_params=pltpu.CompilerParams(use_tc_tiling_on_sc=False),
)(_sc_body)


def kernel(inputs, table):
    idx2d = inputs.reshape(_TOTAL // _SUB, _SUB).astype(jnp.int32)
    enc = _make_enc()
    table_p = jnp.pad(table, ((0, 0), (0, _D)))
    return _sc_gather(idx2d, table_p, enc)
